# Initial kernel scaffold; baseline (speedup 1.0000x reference)
import jax
import jax.numpy as jnp
from jax.experimental import pallas as pl


def kernel(grid, row_pos, col_pos, token_type, ln_gamma, ln_beta):
    raise NotImplementedError("write your pallas kernel here")


# speedup vs baseline: 1.2712x; 1.2712x over previous
import jax
import jax.numpy as jnp
from jax.experimental import pallas as pl

B, F, H, W, C = 8, 8, 48, 48, 256
EPS = 1e-12
ROWS_PER_BLOCK = 8


def _body(g_ref, rp_ref, cp_ref, tt_ref, gm_ref, bt_ref, o_ref):
    g = g_ref[0]
    s = jnp.sum(g, axis=0) * (1.0 / F)
    pos = rp_ref[...][:, None, :] + cp_ref[...][None, :, :]
    e = s + pos + tt_ref[...][0][None, None, :]
    mean = jnp.mean(e, axis=-1, keepdims=True)
    d = e - mean
    var = jnp.mean(d * d, axis=-1, keepdims=True)
    o_ref[0] = d * jax.lax.rsqrt(var + EPS) * gm_ref[0] + bt_ref[0]


def kernel(grid, row_pos, col_pos, token_type, ln_gamma, ln_beta):
    nrow = H // ROWS_PER_BLOCK
    out = pl.pallas_call(
        _body,
        grid=(B, nrow),
        in_specs=[
            pl.BlockSpec((1, F, ROWS_PER_BLOCK, W, C), lambda b, i: (b, 0, i, 0, 0)),
            pl.BlockSpec((ROWS_PER_BLOCK, C), lambda b, i: (i, 0)),
            pl.BlockSpec((W, C), lambda b, i: (0, 0)),
            pl.BlockSpec((1, C), lambda b, i: (0, 0)),
            pl.BlockSpec((1, C), lambda b, i: (0, 0)),
            pl.BlockSpec((1, C), lambda b, i: (0, 0)),
        ],
        out_specs=pl.BlockSpec((1, ROWS_PER_BLOCK, W, C), lambda b, i: (b, i, 0, 0)),
        out_shape=jax.ShapeDtypeStruct((B, H, W, C), jnp.float32),
    )(grid, row_pos, col_pos, token_type,
      ln_gamma.reshape(1, C), ln_beta.reshape(1, C))
    emb = out.reshape(B, H * W, C)
    sampled_indices = jnp.arange(H * W, dtype=jnp.int32)
    return (emb, sampled_indices)


# speedup vs baseline: 1.5355x; 1.2079x over previous
import jax
import jax.numpy as jnp
from jax.experimental import pallas as pl

B, F, H, W, C = 8, 8, 48, 48, 256
EPS = 1e-12
ROWS_PER_BLOCK = 16


def _body(g_ref, rp_ref, cp_ref, tt_ref, gm_ref, bt_ref, o_ref):
    g = g_ref[0]
    s = jnp.sum(g, axis=0) * (1.0 / F)
    pos = rp_ref[...][:, None, :] + cp_ref[...][None, :, :]
    e = s + pos + tt_ref[...][0][None, None, :]
    mean = jnp.mean(e, axis=-1, keepdims=True)
    d = e - mean
    var = jnp.mean(d * d, axis=-1, keepdims=True)
    o_ref[0] = d * jax.lax.rsqrt(var + EPS) * gm_ref[0] + bt_ref[0]


def kernel(grid, row_pos, col_pos, token_type, ln_gamma, ln_beta):
    nrow = H // ROWS_PER_BLOCK
    out = pl.pallas_call(
        _body,
        grid=(B, nrow),
        in_specs=[
            pl.BlockSpec((1, F, ROWS_PER_BLOCK, W, C), lambda b, i: (b, 0, i, 0, 0)),
            pl.BlockSpec((ROWS_PER_BLOCK, C), lambda b, i: (i, 0)),
            pl.BlockSpec((W, C), lambda b, i: (0, 0)),
            pl.BlockSpec((1, C), lambda b, i: (0, 0)),
            pl.BlockSpec((1, C), lambda b, i: (0, 0)),
            pl.BlockSpec((1, C), lambda b, i: (0, 0)),
        ],
        out_specs=pl.BlockSpec((1, ROWS_PER_BLOCK, W, C), lambda b, i: (b, i, 0, 0)),
        out_shape=jax.ShapeDtypeStruct((B, H, W, C), jnp.float32),
    )(grid, row_pos, col_pos, token_type,
      ln_gamma.reshape(1, C), ln_beta.reshape(1, C))
    emb = out.reshape(B, H * W, C)
    sampled_indices = jnp.arange(H * W, dtype=jnp.int32)
    return (emb, sampled_indices)
